# baseline (speedup 1.0000x reference)
import functools

import jax
import jax.numpy as jnp
from jax import lax
from jax.experimental import pallas as pl
from jax.experimental.pallas import tpu as pltpu
from jax.experimental.pallas import tpu_sc as plsc

L = 16
NIN = 8
R = 4


@functools.cache
def _build(batch, z_dim):
    info = plsc.get_sparse_core_info()
    NC, NS = info.num_cores, info.num_subcores
    NW = NC * NS
    rows_per_w = batch // NW
    nblocks = rows_per_w // R
    nchunks = z_dim // L

    mesh = plsc.VectorSubcoreMesh(core_axis_name="c", subcore_axis_name="s")

    @functools.partial(
        pl.kernel,
        mesh=mesh,
        compiler_params=pltpu.CompilerParams(
            needs_layout_passes=False,
            use_tc_tiling_on_sc=True,
        ),
        out_type=jax.ShapeDtypeStruct((batch, z_dim), jnp.float32),
        scratch_types=[
            pltpu.VMEM((z_dim,), jnp.int32),
            *[pltpu.VMEM((R, z_dim), jnp.float32) for _ in range(NIN)],
            *[pltpu.VMEM((R, z_dim), jnp.float32) for _ in range(2)],
            *[pltpu.VMEM((R, z_dim), jnp.float32) for _ in range(2)],
            pltpu.VMEM_SHARED((NS, 2, R, z_dim), jnp.float32),
            *[pltpu.SemaphoreType.DMA for _ in range(NIN + 2 + 2 + 2)],
        ],
    )
    def k(z_hbm, perm_hbm, out_hbm, perm_v, *bufs):
        ins = bufs[0:NIN]
        douts = bufs[NIN : NIN + 2]
        xouts = bufs[NIN + 2 : NIN + 4]
        sp = bufs[NIN + 4]
        isems = bufs[NIN + 5 : 2 * NIN + 5]
        dsems = bufs[2 * NIN + 5 : 2 * NIN + 7]
        xsems = bufs[2 * NIN + 7 : 2 * NIN + 9]
        wsems = bufs[2 * NIN + 9 :]
        sid = lax.axis_index("s")
        wid = sid * NC + lax.axis_index("c")
        base = wid * rows_per_w

        pltpu.sync_copy(perm_hbm, perm_v)

        def start_in(b, q):
            pltpu.async_copy(z_hbm.at[pl.ds(base + b * R, R)], ins[q], isems[q])

        def wait_in(b, q):
            pltpu.make_async_copy(
                z_hbm.at[pl.ds(base + b * R, R)], ins[q], isems[q]
            ).wait()

        def start_direct(b, qd):
            pltpu.async_copy(
                douts[qd], out_hbm.at[pl.ds(base + b * R, R)], dsems[qd]
            )

        def wait_direct(b, qd):
            pltpu.make_async_copy(
                douts[qd], out_hbm.at[pl.ds(base + b * R, R)], dsems[qd]
            ).wait()

        def start_xbar(qx, qs):
            pltpu.async_copy(xouts[qx], sp.at[sid, qs], xsems[qx])

        def wait_xbar(qx, qs):
            pltpu.make_async_copy(xouts[qx], sp.at[sid, qs], xsems[qx]).wait()

        def start_hbm(b, qs):
            pltpu.async_copy(
                sp.at[sid, qs], out_hbm.at[pl.ds(base + b * R, R)], wsems[qs]
            )

        def wait_hbm(b, qs):
            pltpu.make_async_copy(
                sp.at[sid, qs], out_hbm.at[pl.ds(base + b * R, R)], wsems[qs]
            ).wait()

        def compute(qi, out_v):
            in_v = ins[qi]

            @plsc.parallel_loop(0, nchunks, unroll=8)
            def chunk(c):
                idx = perm_v[pl.ds(c * L, L)]
                for r in range(R):
                    row = jnp.full((L,), r, jnp.int32)
                    out_v[r, pl.ds(c * L, L)] = plsc.load_gather(in_v, [row, idx])

        for q in range(NIN):
            start_in(q, q)

        @pl.loop(0, nblocks, step=NIN)
        def body(g):
            for q in range(NIN):
                b = g + q
                qi = q
                j = q // 2
                wait_in(b, qi)

                if q % 2 == 0:
                    qd = j % 2

                    @pl.when(b >= 4)
                    def _():
                        wait_direct(b - 4, qd)

                    compute(qi, douts[qd])
                    start_direct(b, qd)
                else:
                    qx = j % 2
                    qs = j % 2

                    @pl.when(b >= 4)
                    def _():
                        wait_hbm(b - 4, qs)

                    compute(qi, xouts[qx])
                    start_xbar(qx, qs)

                    @pl.when(b >= 3)
                    def _():
                        wait_xbar((j - 1) % 2, (j - 1) % 2)
                        start_hbm(b - 2, (j - 1) % 2)

                @pl.when(b + NIN < nblocks)
                def _():
                    start_in(b + NIN, qi)

        wait_xbar(1, 1)
        start_hbm(nblocks - 1, 1)
        wait_direct(nblocks - 4, 0)
        wait_direct(nblocks - 2, 1)
        wait_hbm(nblocks - 3, 0)
        wait_hbm(nblocks - 1, 1)

    return k


def kernel(z, perm):
    batch, z_dim = z.shape
    k = _build(batch, z_dim)
    return k(z, perm.astype(jnp.int32))


# speedup vs baseline: 1.0533x; 1.0533x over previous
import functools

import jax
import jax.numpy as jnp
from jax import lax
from jax.experimental import pallas as pl
from jax.experimental.pallas import tpu as pltpu
from jax.experimental.pallas import tpu_sc as plsc

L = 16
NIN = 8
NOUT = 2
NSP = 4
R = 4


@functools.cache
def _build(batch, z_dim):
    info = plsc.get_sparse_core_info()
    NC, NS = info.num_cores, info.num_subcores
    NW = NC * NS
    rows_per_w = batch // NW
    nblocks = rows_per_w // R
    nchunks = z_dim // L

    mesh = plsc.VectorSubcoreMesh(core_axis_name="c", subcore_axis_name="s")

    @functools.partial(
        pl.kernel,
        mesh=mesh,
        compiler_params=pltpu.CompilerParams(
            needs_layout_passes=False,
            use_tc_tiling_on_sc=True,
        ),
        out_type=jax.ShapeDtypeStruct((batch, z_dim), jnp.float32),
        scratch_types=[
            pltpu.VMEM((z_dim,), jnp.int32),
            *[pltpu.VMEM((R, z_dim), jnp.float32) for _ in range(NIN + NOUT)],
            pltpu.VMEM_SHARED((NS, NSP, R, z_dim), jnp.float32),
            *[pltpu.SemaphoreType.DMA for _ in range(NIN + NOUT + NSP)],
        ],
    )
    def k(z_hbm, perm_hbm, out_hbm, perm_v, *bufs):
        ins = bufs[:NIN]
        outs = bufs[NIN : NIN + NOUT]
        sp = bufs[NIN + NOUT]
        isems = bufs[NIN + NOUT + 1 : 2 * NIN + NOUT + 1]
        xsems = bufs[2 * NIN + NOUT + 1 : 2 * NIN + 2 * NOUT + 1]
        wsems = bufs[2 * NIN + 2 * NOUT + 1 :]
        sid = lax.axis_index("s")
        wid = sid * NC + lax.axis_index("c")
        base = wid * rows_per_w

        pltpu.sync_copy(perm_hbm, perm_v)

        def start_in(b, q):
            pltpu.async_copy(z_hbm.at[pl.ds(base + b * R, R)], ins[q], isems[q])

        def wait_in(b, q):
            pltpu.make_async_copy(
                z_hbm.at[pl.ds(base + b * R, R)], ins[q], isems[q]
            ).wait()

        def start_xbar(qo, qs):
            pltpu.async_copy(outs[qo], sp.at[sid, qs], xsems[qo])

        def wait_xbar(qo, qs):
            pltpu.make_async_copy(outs[qo], sp.at[sid, qs], xsems[qo]).wait()

        def start_hbm(b, qs):
            pltpu.async_copy(
                sp.at[sid, qs], out_hbm.at[pl.ds(base + b * R, R)], wsems[qs]
            )

        def wait_hbm(b, qs):
            pltpu.make_async_copy(
                sp.at[sid, qs], out_hbm.at[pl.ds(base + b * R, R)], wsems[qs]
            ).wait()

        def compute(qi, qo):
            in_v, out_v = ins[qi], outs[qo]

            @plsc.parallel_loop(0, nchunks, unroll=8)
            def chunk(c):
                idx = perm_v[pl.ds(c * L, L)]
                for r in range(R):
                    row = jnp.full((L,), r, jnp.int32)
                    out_v[r, pl.ds(c * L, L)] = plsc.load_gather(in_v, [row, idx])

        for q in range(NIN):
            start_in(q, q)

        @pl.loop(0, nblocks, step=NIN)
        def body(g):
            for q in range(NIN):
                b = g + q
                qi = q
                qo = q % NOUT
                qs = q % NSP
                wait_in(b, qi)
                compute(qi, qo)

                @pl.when(b >= NSP)
                def _():
                    wait_hbm(b - NSP, qs)

                start_xbar(qo, qs)

                @pl.when(b >= 1)
                def _():
                    wait_xbar((q - 1) % NOUT, (q - 1) % NSP)
                    start_hbm(b - 1, (q - 1) % NSP)

                @pl.when(b + NIN < nblocks)
                def _():
                    start_in(b + NIN, qi)

        last = nblocks - 1
        wait_xbar(last % NOUT, last % NSP)
        start_hbm(last, last % NSP)
        for d in range(NSP):
            wait_hbm(nblocks - NSP + d, (nblocks - NSP + d) % NSP)

    return k


def kernel(z, perm):
    batch, z_dim = z.shape
    k = _build(batch, z_dim)
    return k(z, perm.astype(jnp.int32))
